# Initial kernel scaffold; baseline (speedup 1.0000x reference)
import jax
import jax.numpy as jnp
from jax.experimental import pallas as pl


def kernel(x, start_token, end_token, tok_table, pos_table):
    raise NotImplementedError("write your pallas kernel here")


# speedup vs baseline: 2.4363x; 2.4363x over previous
import functools

import jax
import jax.numpy as jnp
from jax import lax
from jax.experimental import pallas as pl
from jax.experimental.pallas import tpu as pltpu
from jax.experimental.pallas import tpu_sc as plsc

B, T, V, D = 4096, 200, 1000, 64
VA = V + T
LANES = 16
JJ = D // LANES

_info = plsc.get_sparse_core_info()
NC, NS = _info.num_cores, _info.num_subcores
NW = NC * NS
SENT_PER_W = B // NW
CHUNK_WORDS = T * D
NGROUP = (T + LANES - 1) // LANES
TPAD = NGROUP * LANES


@functools.partial(
    pl.kernel,
    out_type=jax.ShapeDtypeStruct((B * T * D,), jnp.float32),
    mesh=plsc.VectorSubcoreMesh(core_axis_name="c", subcore_axis_name="s"),
    scratch_types=[
        pltpu.VMEM((VA * D,), jnp.float32),
        pltpu.VMEM((TPAD * D,), jnp.float32),
        pltpu.VMEM((TPAD,), jnp.int32),
        pltpu.VMEM((TPAD * D,), jnp.float32),
    ],
)
def _emb_kernel(aug_hbm, x_hbm, pos_hbm, out_hbm, tok_v, pos_v, x_v, rows_v):
    wid = lax.axis_index("s") * NC + lax.axis_index("c")

    pltpu.sync_copy(aug_hbm, tok_v)
    pltpu.sync_copy(pos_hbm, pos_v.at[pl.ds(0, T * D)])

    lane_iota = lax.iota(jnp.int32, LANES)

    def per_sentence(c, carry):
        b = wid * SENT_PER_W + c
        pltpu.sync_copy(x_hbm.at[pl.ds(b * T, T)], x_v.at[pl.ds(0, T)])

        def per_group(g, carry2):
            xg = x_v[pl.ds(g * LANES, LANES)]
            tvec = g * LANES + lane_iota
            idxv = jnp.where(xg == 2, V + tvec, xg)
            idxv = jnp.clip(idxv, 0, VA - 1)
            offv = idxv * D
            for i in range(LANES):
                off = offv[i]
                rbase = (g * LANES + i) * D
                for jj in range(JJ):
                    rows_v[pl.ds(rbase + jj * LANES, LANES)] = (
                        tok_v[pl.ds(off + jj * LANES, LANES)]
                        + pos_v[pl.ds(rbase + jj * LANES, LANES)]
                    )
            return carry2

        lax.fori_loop(0, NGROUP, per_group, 0)
        pltpu.sync_copy(
            rows_v.at[pl.ds(0, CHUNK_WORDS)],
            out_hbm.at[pl.ds(b * CHUNK_WORDS, CHUNK_WORDS)],
        )
        return carry

    lax.fori_loop(0, SENT_PER_W, per_sentence, 0)


def kernel(x, start_token, end_token, tok_table, pos_table):
    aug = jnp.concatenate(
        [tok_table, jnp.float32(-5.0) - pos_table], axis=0
    ).reshape(-1)
    out_flat = _emb_kernel(
        aug, x.reshape(-1).astype(jnp.int32), pos_table.reshape(-1)
    )
    return out_flat.reshape(B, T, D)


# speedup vs baseline: 2.4895x; 1.0219x over previous
import functools

import jax
import jax.numpy as jnp
from jax import lax
from jax.experimental import pallas as pl
from jax.experimental.pallas import tpu as pltpu
from jax.experimental.pallas import tpu_sc as plsc

B, T, V, D = 4096, 200, 1000, 64
VA = V + T
LANES = 16
JJ = D // LANES

_info = plsc.get_sparse_core_info()
NC, NS = _info.num_cores, _info.num_subcores
NW = NC * NS
SENT_PER_W = B // NW
NGROUP = (T + LANES - 1) // LANES
TPAD = NGROUP * LANES
G0, G1 = 128, TPAD - 128


@functools.partial(
    pl.kernel,
    out_type=jax.ShapeDtypeStruct((B * T, D), jnp.float32),
    mesh=plsc.VectorSubcoreMesh(core_axis_name="c", subcore_axis_name="s"),
    compiler_params=pltpu.CompilerParams(use_tc_tiling_on_sc=False),
    scratch_types=[
        pltpu.VMEM((TPAD * D,), jnp.float32),
        pltpu.VMEM((TPAD,), jnp.int32),
        pltpu.VMEM((TPAD,), jnp.int32),
        pltpu.VMEM((TPAD,), jnp.int32),
        pltpu.VMEM((TPAD,), jnp.int32),
        pltpu.VMEM((TPAD, D), jnp.float32),
        pltpu.VMEM((TPAD, D), jnp.float32),
        pltpu.SemaphoreType.DMA,
        pltpu.SemaphoreType.DMA,
        pltpu.SemaphoreType.DMA,
        pltpu.SemaphoreType.DMA,
        pltpu.SemaphoreType.DMA,
        pltpu.SemaphoreType.DMA,
    ],
)
def _emb_kernel(aug_hbm, x_hbm, pos_hbm, out_hbm,
                pos_v, x0, x1, i0, i1, r0, r1,
                sx0, sx1, sg0, sg1, so0, so1):
    wid = lax.axis_index("s") * NC + lax.axis_index("c")
    base = wid * SENT_PER_W

    xs = (x0, x1)
    idxs = (i0, i1)
    rows = (r0, r1)
    sxs = (sx0, sx1)
    sgs = (sg0, sg1)
    sos = (so0, so1)

    pltpu.sync_copy(pos_hbm, pos_v.at[pl.ds(0, T * D)])

    lane_iota = lax.iota(jnp.int32, LANES)

    def fire_x(c, slot):
        pltpu.async_copy(
            x_hbm.at[pl.ds((base + c) * T, T)], xs[slot].at[pl.ds(0, T)],
            sxs[slot])

    def wait_x(slot):
        pltpu.make_async_copy(
            x_hbm.at[pl.ds(0, T)], xs[slot].at[pl.ds(0, T)],
            sxs[slot]).wait()

    def transform(slot):
        xv, iv = xs[slot], idxs[slot]

        def grp(g, carry):
            xg = xv[pl.ds(g * LANES, LANES)]
            tvec = g * LANES + lane_iota
            idxg = jnp.where(xg == 2, V + tvec, xg)
            iv[pl.ds(g * LANES, LANES)] = jnp.clip(idxg, 0, VA - 1)
            return carry

        lax.fori_loop(0, NGROUP, grp, 0)

    def fire_gather(slot):
        iv, rv = idxs[slot], rows[slot]
        pltpu.async_copy(
            aug_hbm.at[iv.at[pl.ds(0, G0)]], rv.at[pl.ds(0, G0)], sgs[slot])
        pltpu.async_copy(
            aug_hbm.at[iv.at[pl.ds(G0, G1)]], rv.at[pl.ds(G0, G1)], sgs[slot])

    def wait_gather(slot):
        iv, rv = idxs[slot], rows[slot]
        pltpu.make_async_copy(
            aug_hbm.at[iv.at[pl.ds(0, G0)]], rv.at[pl.ds(0, G0)],
            sgs[slot]).wait()
        pltpu.make_async_copy(
            aug_hbm.at[iv.at[pl.ds(G0, G1)]], rv.at[pl.ds(G0, G1)],
            sgs[slot]).wait()

    def add_pos(slot):
        rv = rows[slot]

        def row(r, carry):
            rbase = r * D
            for jj in range(JJ):
                sl = pl.ds(jj * LANES, LANES)
                rv[r, sl] = rv[r, sl] + pos_v[pl.ds(rbase + jj * LANES, LANES)]
            return carry

        lax.fori_loop(0, T, row, 0)

    def fire_out(c, slot):
        pltpu.async_copy(
            rows[slot].at[pl.ds(0, T)],
            out_hbm.at[pl.ds((base + c) * T, T)], sos[slot])

    def wait_out(slot):
        pltpu.make_async_copy(
            rows[slot].at[pl.ds(0, T)],
            out_hbm.at[pl.ds(0, T)], sos[slot]).wait()

    fire_x(0, 0)
    fire_x(1, 1)
    wait_x(0)
    transform(0)
    fire_x(2, 0)
    fire_gather(0)

    def step(c, carry):
        s0 = lax.rem(c, 2)

        @pl.when(c + 1 < SENT_PER_W)
        def _():
            for slot in (0, 1):

                @pl.when(s0 != slot)
                def _():
                    wait_x(slot)
                    transform(slot)

                    @pl.when(c + 3 < SENT_PER_W)
                    def _():
                        fire_x(c + 3, slot)

                    @pl.when(c >= 1)
                    def _():
                        wait_out(slot)

                    fire_gather(slot)

        for slot in (0, 1):

            @pl.when(s0 == slot)
            def _():
                wait_gather(slot)
                add_pos(slot)
                fire_out(c, slot)

        return carry

    lax.fori_loop(0, SENT_PER_W, step, 0)

    wait_out(0)
    wait_out(1)


def kernel(x, start_token, end_token, tok_table, pos_table):
    aug = jnp.concatenate([tok_table, jnp.float32(-5.0) - pos_table], axis=0)
    out = _emb_kernel(
        aug, x.reshape(-1).astype(jnp.int32), pos_table.reshape(-1)
    )
    return out.reshape(B, T, D)


# speedup vs baseline: 2.4943x; 1.0019x over previous
import functools

import jax
import jax.numpy as jnp
from jax import lax
from jax.experimental import pallas as pl
from jax.experimental.pallas import tpu as pltpu
from jax.experimental.pallas import tpu_sc as plsc

B, T, V, D = 4096, 200, 1000, 64
VA = V + T
LANES = 16
JJ = D // LANES

_info = plsc.get_sparse_core_info()
NC, NS = _info.num_cores, _info.num_subcores
NW = NC * NS
SENT_PER_W = B // NW
NGROUP = (T + LANES - 1) // LANES
TPAD = NGROUP * LANES
G0, G1 = 128, TPAD - 128


@functools.partial(
    pl.kernel,
    out_type=jax.ShapeDtypeStruct((B * T, D), jnp.float32),
    mesh=plsc.VectorSubcoreMesh(core_axis_name="c", subcore_axis_name="s"),
    compiler_params=pltpu.CompilerParams(use_tc_tiling_on_sc=False),
    scratch_types=[
        pltpu.VMEM((TPAD * D,), jnp.float32),
        pltpu.VMEM((TPAD,), jnp.int32),
        pltpu.VMEM((TPAD,), jnp.int32),
        pltpu.VMEM((TPAD,), jnp.int32),
        pltpu.VMEM((TPAD,), jnp.int32),
        pltpu.VMEM((TPAD, D), jnp.float32),
        pltpu.VMEM((TPAD, D), jnp.float32),
        pltpu.SemaphoreType.DMA,
        pltpu.SemaphoreType.DMA,
        pltpu.SemaphoreType.DMA,
        pltpu.SemaphoreType.DMA,
        pltpu.SemaphoreType.DMA,
        pltpu.SemaphoreType.DMA,
    ],
)
def _emb_kernel(aug_hbm, x_hbm, pos_hbm, out_hbm,
                pos_v, x0, x1, i0, i1, r0, r1,
                sx0, sx1, sg0, sg1, so0, so1):
    wid = lax.axis_index("s") * NC + lax.axis_index("c")
    base = wid * SENT_PER_W

    xs = (x0, x1)
    idxs = (i0, i1)
    rows = (r0, r1)
    sxs = (sx0, sx1)
    sgs = (sg0, sg1)
    sos = (so0, so1)

    pltpu.sync_copy(pos_hbm, pos_v.at[pl.ds(0, T * D)])

    lane_iota = lax.iota(jnp.int32, LANES)

    def fire_x(c, slot):
        pltpu.async_copy(
            x_hbm.at[pl.ds((base + c) * T, T)], xs[slot].at[pl.ds(0, T)],
            sxs[slot])

    def wait_x(slot):
        pltpu.make_async_copy(
            x_hbm.at[pl.ds(0, T)], xs[slot].at[pl.ds(0, T)],
            sxs[slot]).wait()

    def transform(slot):
        xv, iv = xs[slot], idxs[slot]

        def grp(g, carry):
            xg = xv[pl.ds(g * LANES, LANES)]
            tvec = g * LANES + lane_iota
            idxg = jnp.where(xg == 2, V + tvec, xg)
            iv[pl.ds(g * LANES, LANES)] = jnp.clip(idxg, 0, VA - 1)
            return carry

        lax.fori_loop(0, NGROUP, grp, 0)

    def fire_gather(slot):
        iv, rv = idxs[slot], rows[slot]
        pltpu.async_copy(
            aug_hbm.at[iv.at[pl.ds(0, G0)]], rv.at[pl.ds(0, G0)], sgs[slot])
        pltpu.async_copy(
            aug_hbm.at[iv.at[pl.ds(G0, G1)]], rv.at[pl.ds(G0, G1)], sgs[slot])

    def wait_gather(slot):
        iv, rv = idxs[slot], rows[slot]
        pltpu.make_async_copy(
            aug_hbm.at[iv.at[pl.ds(0, G0)]], rv.at[pl.ds(0, G0)],
            sgs[slot]).wait()
        pltpu.make_async_copy(
            aug_hbm.at[iv.at[pl.ds(G0, G1)]], rv.at[pl.ds(G0, G1)],
            sgs[slot]).wait()

    def add_pos(slot):
        rv = rows[slot]

        def row(r, carry):
            rbase = r * D
            for jj in range(JJ):
                sl = pl.ds(jj * LANES, LANES)
                plsc.addupdate(
                    rv.at[r, sl], pos_v[pl.ds(rbase + jj * LANES, LANES)])
            return carry

        lax.fori_loop(0, T, row, 0, unroll=8)

    def fire_out(c, slot):
        pltpu.async_copy(
            rows[slot].at[pl.ds(0, T)],
            out_hbm.at[pl.ds((base + c) * T, T)], sos[slot])

    def wait_out(slot):
        pltpu.make_async_copy(
            rows[slot].at[pl.ds(0, T)],
            out_hbm.at[pl.ds(0, T)], sos[slot]).wait()

    fire_x(0, 0)
    fire_x(1, 1)
    wait_x(0)
    transform(0)
    fire_x(2, 0)
    fire_gather(0)

    def step(c, carry):
        s0 = lax.rem(c, 2)

        @pl.when(c + 1 < SENT_PER_W)
        def _():
            for slot in (0, 1):

                @pl.when(s0 != slot)
                def _():
                    wait_x(slot)
                    transform(slot)

                    @pl.when(c + 3 < SENT_PER_W)
                    def _():
                        fire_x(c + 3, slot)

                    @pl.when(c >= 1)
                    def _():
                        wait_out(slot)

                    fire_gather(slot)

        for slot in (0, 1):

            @pl.when(s0 == slot)
            def _():
                wait_gather(slot)
                add_pos(slot)
                fire_out(c, slot)

        return carry

    lax.fori_loop(0, SENT_PER_W, step, 0)

    wait_out(0)
    wait_out(1)


def kernel(x, start_token, end_token, tok_table, pos_table):
    aug = jnp.concatenate([tok_table, jnp.float32(-5.0) - pos_table], axis=0)
    out = _emb_kernel(
        aug, x.reshape(-1).astype(jnp.int32), pos_table.reshape(-1)
    )
    return out.reshape(B, T, D)


# speedup vs baseline: 3.6810x; 1.4758x over previous
import functools

import jax
import jax.numpy as jnp
from jax import lax
from jax.experimental import pallas as pl
from jax.experimental.pallas import tpu as pltpu
from jax.experimental.pallas import tpu_sc as plsc

B, T, V, D = 4096, 200, 1000, 64
VA = V + T
LANES = 16
JJ = D // LANES

_info = plsc.get_sparse_core_info()
NC, NS = _info.num_cores, _info.num_subcores
NW = NC * NS
SENT_PER_W = B // NW
WORDS_PER_W = SENT_PER_W * T
NGRP = WORDS_PER_W // LANES
G0 = 128
G1 = T - G0


@functools.partial(
    pl.kernel,
    out_type=jax.ShapeDtypeStruct((B * T, D), jnp.float32),
    mesh=plsc.VectorSubcoreMesh(core_axis_name="c", subcore_axis_name="s"),
    compiler_params=pltpu.CompilerParams(use_tc_tiling_on_sc=False),
    scratch_types=[
        pltpu.VMEM((T * D,), jnp.float32),
        pltpu.VMEM((WORDS_PER_W,), jnp.int32),
        pltpu.VMEM((T, D), jnp.float32),
        pltpu.VMEM((T, D), jnp.float32),
        pltpu.VMEM((T, D), jnp.float32),
        pltpu.VMEM((T, D), jnp.float32),
        pltpu.SemaphoreType.DMA,
        pltpu.SemaphoreType.DMA,
        pltpu.SemaphoreType.DMA,
        pltpu.SemaphoreType.DMA,
        pltpu.SemaphoreType.DMA,
        pltpu.SemaphoreType.DMA,
        pltpu.SemaphoreType.DMA,
        pltpu.SemaphoreType.DMA,
        pltpu.SemaphoreType.DMA,
    ],
)
def _emb_kernel(aug_hbm, x_hbm, pos_hbm, out_hbm,
                pos_v, idx_v, r0, r1, r2, r3,
                si, sg0, sg1, sg2, sg3, so0, so1, so2, so3):
    wid = lax.axis_index("s") * NC + lax.axis_index("c")
    base = wid * SENT_PER_W

    rows = (r0, r1, r2, r3)
    sgs = (sg0, sg1, sg2, sg3)
    sos = (so0, so1, so2, so3)

    pltpu.async_copy(x_hbm.at[pl.ds(base * T, WORDS_PER_W)], idx_v, si)
    pltpu.sync_copy(pos_hbm, pos_v)
    pltpu.make_async_copy(
        x_hbm.at[pl.ds(0, WORDS_PER_W)], idx_v, si).wait()

    lane_iota = lax.iota(jnp.int32, LANES)

    def grp(k, carry):
        xg = idx_v[pl.ds(k * LANES, LANES)]
        tv = lax.rem(k * LANES + lane_iota, T)
        idxg = jnp.where(xg == 2, V + tv, xg)
        idx_v[pl.ds(k * LANES, LANES)] = jnp.clip(idxg, 0, VA - 1)
        return carry

    lax.fori_loop(0, NGRP, grp, 0, unroll=4)

    def fire_gather(c, slot):
        rv = rows[slot]
        pltpu.async_copy(
            aug_hbm.at[idx_v.at[pl.ds(c * T, G0)]],
            rv.at[pl.ds(0, G0)], sgs[slot])
        pltpu.async_copy(
            aug_hbm.at[idx_v.at[pl.ds(c * T + G0, G1)]],
            rv.at[pl.ds(G0, G1)], sgs[slot])

    def wait_gather(slot):
        rv = rows[slot]
        pltpu.make_async_copy(
            aug_hbm.at[idx_v.at[pl.ds(0, G0)]],
            rv.at[pl.ds(0, G0)], sgs[slot]).wait()
        pltpu.make_async_copy(
            aug_hbm.at[idx_v.at[pl.ds(0, G1)]],
            rv.at[pl.ds(G0, G1)], sgs[slot]).wait()

    def add_pos(slot):
        rv = rows[slot]

        def row(r, carry):
            rbase = r * D
            for jj in range(JJ):
                sl = pl.ds(jj * LANES, LANES)
                plsc.addupdate(
                    rv.at[r, sl], pos_v[pl.ds(rbase + jj * LANES, LANES)])
            return carry

        lax.fori_loop(0, T, row, 0, unroll=4)

    def fire_out(c, slot):
        pltpu.async_copy(
            rows[slot], out_hbm.at[pl.ds((base + c) * T, T)], sos[slot])

    def wait_out(slot):
        pltpu.make_async_copy(
            rows[slot], out_hbm.at[pl.ds(0, T)], sos[slot]).wait()

    fire_gather(0, 0)

    def super_step(g, carry):
        for slot in (0, 1, 2, 3):
            c = g * 4 + slot
            nslot = (slot + 1) % 4

            @pl.when(c >= 3)
            def _():
                wait_out(nslot)

            @pl.when(c + 1 < SENT_PER_W)
            def _():
                fire_gather(c + 1, nslot)

            wait_gather(slot)
            add_pos(slot)
            fire_out(c, slot)
        return carry

    lax.fori_loop(0, SENT_PER_W // 4, super_step, 0)

    wait_out(1)
    wait_out(2)
    wait_out(3)


def kernel(x, start_token, end_token, tok_table, pos_table):
    aug = jnp.concatenate([tok_table, jnp.float32(-5.0) - pos_table], axis=0)
    out = _emb_kernel(
        aug, x.reshape(-1).astype(jnp.int32), pos_table.reshape(-1)
    )
    return out.reshape(B, T, D)


# speedup vs baseline: 3.6850x; 1.0011x over previous
import functools

import jax
import jax.numpy as jnp
from jax import lax
from jax.experimental import pallas as pl
from jax.experimental.pallas import tpu as pltpu
from jax.experimental.pallas import tpu_sc as plsc

B, T, V, D = 4096, 200, 1000, 64
VA = V + T
LANES = 16
JJ = D // LANES

_info = plsc.get_sparse_core_info()
NC, NS = _info.num_cores, _info.num_subcores
NW = NC * NS
SENT_PER_W = B // NW
WORDS_PER_W = SENT_PER_W * T
NGRP = WORDS_PER_W // LANES
G0 = 128
G1 = T - G0


@functools.partial(
    pl.kernel,
    out_type=jax.ShapeDtypeStruct((B * T, D), jnp.float32),
    mesh=plsc.VectorSubcoreMesh(core_axis_name="c", subcore_axis_name="s"),
    compiler_params=pltpu.CompilerParams(use_tc_tiling_on_sc=False),
    scratch_types=[
        pltpu.VMEM((T * D,), jnp.float32),
        pltpu.VMEM((WORDS_PER_W,), jnp.int32),
        pltpu.VMEM((T, D), jnp.float32),
        pltpu.VMEM((T, D), jnp.float32),
        pltpu.VMEM((T, D), jnp.float32),
        pltpu.VMEM((T, D), jnp.float32),
        pltpu.SemaphoreType.DMA,
        pltpu.SemaphoreType.DMA,
        pltpu.SemaphoreType.DMA,
        pltpu.SemaphoreType.DMA,
        pltpu.SemaphoreType.DMA,
        pltpu.SemaphoreType.DMA,
        pltpu.SemaphoreType.DMA,
        pltpu.SemaphoreType.DMA,
        pltpu.SemaphoreType.DMA,
    ],
)
def _emb_kernel(aug_hbm, x_hbm, pos_hbm, out_hbm,
                pos_v, idx_v, r0, r1, r2, r3,
                si, sg0, sg1, sg2, sg3, so0, so1, so2, so3):
    wid = lax.axis_index("s") * NC + lax.axis_index("c")
    base = wid * SENT_PER_W

    rows = (r0, r1, r2, r3)
    sgs = (sg0, sg1, sg2, sg3)
    sos = (so0, so1, so2, so3)

    pltpu.async_copy(x_hbm.at[pl.ds(base * T, WORDS_PER_W)], idx_v, si)
    pltpu.sync_copy(pos_hbm, pos_v)
    pltpu.make_async_copy(
        x_hbm.at[pl.ds(0, WORDS_PER_W)], idx_v, si).wait()

    lane_iota = lax.iota(jnp.int32, LANES)

    def grp(k, carry):
        xg = idx_v[pl.ds(k * LANES, LANES)]
        tv = lax.rem(k * LANES + lane_iota, T)
        idxg = jnp.where(xg == 2, V + tv, xg)
        idx_v[pl.ds(k * LANES, LANES)] = jnp.clip(idxg, 0, VA - 1)
        return carry

    lax.fori_loop(0, NGRP, grp, 0, unroll=4)

    def fire_gather(c, slot):
        rv = rows[slot]
        pltpu.async_copy(
            aug_hbm.at[idx_v.at[pl.ds(c * T, G0)]],
            rv.at[pl.ds(0, G0)], sgs[slot])
        pltpu.async_copy(
            aug_hbm.at[idx_v.at[pl.ds(c * T + G0, G1)]],
            rv.at[pl.ds(G0, G1)], sgs[slot])

    def wait_gather(slot):
        rv = rows[slot]
        pltpu.make_async_copy(
            aug_hbm.at[idx_v.at[pl.ds(0, G0)]],
            rv.at[pl.ds(0, G0)], sgs[slot]).wait()
        pltpu.make_async_copy(
            aug_hbm.at[idx_v.at[pl.ds(0, G1)]],
            rv.at[pl.ds(G0, G1)], sgs[slot]).wait()

    def add_pos(slot):
        rv = rows[slot]

        def row(r, carry):
            rbase = r * D
            for jj in range(JJ):
                sl = pl.ds(jj * LANES, LANES)
                plsc.addupdate(
                    rv.at[r, sl], pos_v[pl.ds(rbase + jj * LANES, LANES)])
            return carry

        lax.fori_loop(0, T, row, 0, unroll=4)

    def fire_out(c, slot):
        pltpu.async_copy(
            rows[slot], out_hbm.at[pl.ds((base + c) * T, T)], sos[slot])

    def wait_out(slot):
        pltpu.make_async_copy(
            rows[slot], out_hbm.at[pl.ds(0, T)], sos[slot]).wait()

    fire_gather(0, 0)
    fire_gather(1, 1)

    def super_step(g, carry):
        for slot in (0, 1, 2, 3):
            c = g * 4 + slot
            nslot = (slot + 2) % 4

            @pl.when(c >= 2)
            def _():
                wait_out(nslot)

            @pl.when(c + 2 < SENT_PER_W)
            def _():
                fire_gather(c + 2, nslot)

            wait_gather(slot)
            add_pos(slot)
            fire_out(c, slot)
        return carry

    lax.fori_loop(0, SENT_PER_W // 4, super_step, 0)

    wait_out(2)
    wait_out(3)


def kernel(x, start_token, end_token, tok_table, pos_table):
    aug = jnp.concatenate([tok_table, jnp.float32(-5.0) - pos_table], axis=0)
    out = _emb_kernel(
        aug, x.reshape(-1).astype(jnp.int32), pos_table.reshape(-1)
    )
    return out.reshape(B, T, D)
